# speedup vs baseline: 8.3096x; 8.3096x over previous
import jax
import jax.numpy as jnp
from jax.experimental import pallas as pl
from jax.experimental.pallas import tpu as pltpu
from jax.experimental.pallas import tpu_sc as plsc

_EPS = 1e-12

_VOCAB = 100000
_D = 128
_BATCH = 4096
_SEQ = 200
_N = _BATCH * _SEQ

_GW = 128


def _sc_gather(W, ids_flat):
  mesh = plsc.VectorSubcoreMesh(core_axis_name="c", subcore_axis_name="s")

  @pl.kernel(
      out_type=jax.ShapeDtypeStruct((_N, _D), jnp.float32),
      mesh=mesh,
  )
  def gather_kernel(w_hbm, i_hbm, o_hbm):
    def body(i_vmem, o_vmem):
      pltpu.sync_copy(w_hbm.at[i_vmem.at[0]], o_vmem)

    pltpu.emit_pipeline(
        body,
        grid=(_N // _GW,),
        in_specs=[pl.BlockSpec((1, _GW), index_map=lambda i: (0, i))],
        out_specs=[pl.BlockSpec((_GW, _D), index_map=lambda i: (i, 0))],
        core_axis_name=("c", "s"),
        dimension_semantics=(pltpu.PARALLEL,),
    )(i_hbm, o_hbm)

  return gather_kernel(W, ids_flat)


_BB = 16


def _ln_body(emb_ref, p_ref, g_ref, b_ref, o_ref):
  x = emb_ref[...] + p_ref[...][None, :, :]
  m = jnp.mean(x, axis=-1, keepdims=True)
  d = x - m
  v = jnp.mean(d * d, axis=-1, keepdims=True)
  scale = jax.lax.rsqrt(v + _EPS)
  o_ref[...] = d * scale * g_ref[0][None, None, :] + b_ref[0][None, None, :]


def _tc_ln(emb, W, gamma, beta):
  return pl.pallas_call(
      _ln_body,
      grid=(_BATCH // _BB,),
      in_specs=[
          pl.BlockSpec((_BB, _SEQ, _D), lambda i: (i, 0, 0)),
          pl.BlockSpec((_SEQ, _D), lambda i: (0, 0)),
          pl.BlockSpec((1, _D), lambda i: (0, 0)),
          pl.BlockSpec((1, _D), lambda i: (0, 0)),
      ],
      out_specs=pl.BlockSpec((_BB, _SEQ, _D), lambda i: (i, 0, 0)),
      out_shape=jax.ShapeDtypeStruct((_BATCH, _SEQ, _D), jnp.float32),
  )(emb, W, gamma, beta)


def kernel(input_ids, W, gamma, beta):
  ids_flat = input_ids.reshape(1, _N).astype(jnp.int32)
  gathered = _sc_gather(W, ids_flat)
  emb = gathered.reshape(_BATCH, _SEQ, _D)
  g2 = gamma.reshape(1, _D)
  b2 = beta.reshape(1, _D)
  return _tc_ln(emb, W, g2, b2)
